# baseline (speedup 1.0000x reference)
import functools

import jax
import jax.numpy as jnp
from jax import lax
from jax.experimental import pallas as pl
from jax.experimental.pallas import tpu as pltpu
from jax.experimental.pallas import tpu_sc as plsc

B = 16384
K = 20
D = 16
NC = 2
NS = 16
NW = NC * NS
BPW = B // NW
C = 128
NCHUNK = BPW // C


def _sc_body(center_hbm, context_hbm, negflat_hbm, cw_hbm, ctw_hbm,
             pos_out, neg_out,
             cidx_v, pidx_v, nidx_v, vc_v, vp_v, vn_v, pos_v, neg_v,
             isem, csem, psem, nsem):
    wid = lax.axis_index("s") * NC + lax.axis_index("c")
    iota = lax.iota(jnp.int32, 16)

    for c in range(NCHUNK):
        base = wid * BPW + c * C

        pltpu.sync_copy(center_hbm.at[pl.ds(base, C)], cidx_v)
        pltpu.sync_copy(context_hbm.at[pl.ds(base, C)], pidx_v)
        idx_copies = [
            pltpu.async_copy(negflat_hbm.at[pl.ds(base * K + j * C, C)],
                             nidx_v.at[j], isem)
            for j in range(K)
        ]
        cc = pltpu.async_copy(cw_hbm.at[cidx_v], vc_v, csem)
        pc = pltpu.async_copy(ctw_hbm.at[pidx_v], vp_v, psem)
        for cp in idx_copies:
            cp.wait()
        ncopies = [
            pltpu.async_copy(ctw_hbm.at[nidx_v.at[j]],
                             vn_v.at[pl.ds(j * C, C)], nsem)
            for j in range(K)
        ]
        cc.wait()
        pc.wait()
        for cp in ncopies:
            cp.wait()

        def group(g, _):
            rows = iota + g * 16
            rowsk = rows * K
            pos_acc = jnp.zeros((16,), jnp.float32)
            neg_accs = [jnp.zeros((16,), jnp.float32) for _ in range(K)]
            for d in range(D):
                dsplat = jnp.full((16,), d, jnp.int32)
                vcg = plsc.load_gather(vc_v, [rows, dsplat])
                vpg = plsc.load_gather(vp_v, [rows, dsplat])
                pos_acc = pos_acc + vcg * vpg
                for k in range(K):
                    vng = plsc.load_gather(vn_v, [rowsk + k, dsplat])
                    neg_accs[k] = neg_accs[k] + vng * vcg
            pos_v[pl.ds(g * 16, 16)] = pos_acc
            for k in range(K):
                plsc.store_scatter(neg_v, [rowsk + k], neg_accs[k])
            return 0

        lax.fori_loop(0, C // 16, group, 0)

        pltpu.sync_copy(pos_v, pos_out.at[pl.ds(base, C)])
        pltpu.sync_copy(neg_v, neg_out.at[pl.ds(base * K, C * K)])


@jax.jit
def _sc_dots(center, context, negflat, cw, ctw):
    mesh = plsc.VectorSubcoreMesh(core_axis_name="c", subcore_axis_name="s")
    f = functools.partial(
        pl.kernel,
        out_type=(jax.ShapeDtypeStruct((B,), jnp.float32),
                  jax.ShapeDtypeStruct((B * K,), jnp.float32)),
        mesh=mesh,
        scratch_types=[
            pltpu.VMEM((C,), jnp.int32),
            pltpu.VMEM((C,), jnp.int32),
            pltpu.VMEM((K, C), jnp.int32),
            pltpu.VMEM((C, D), jnp.float32),
            pltpu.VMEM((C, D), jnp.float32),
            pltpu.VMEM((C * K, D), jnp.float32),
            pltpu.VMEM((C,), jnp.float32),
            pltpu.VMEM((C * K,), jnp.float32),
            pltpu.SemaphoreType.DMA,
            pltpu.SemaphoreType.DMA,
            pltpu.SemaphoreType.DMA,
            pltpu.SemaphoreType.DMA,
        ],
        compiler_params=pltpu.CompilerParams(needs_layout_passes=False,
                                             use_tc_tiling_on_sc=False),
    )(_sc_body)
    return f(center, context, negflat, cw, ctw)


def _loss_body(pos_ref, neg_ref, out_ref):
    pos = pos_ref[...]
    neg = neg_ref[...]
    sp = 1.0 / (1.0 + jnp.exp(-pos))
    sn = 1.0 / (1.0 + jnp.exp(neg))
    t1 = -jnp.sum(jnp.log(sp + 1e-8))
    t2 = -jnp.sum(jnp.log(sn + 1e-8))
    out_ref[0, 0] = (t1 + t2) / B


@jax.jit
def _loss(pos, neg):
    return pl.pallas_call(
        _loss_body,
        out_shape=jax.ShapeDtypeStruct((1, 1), jnp.float32),
        in_specs=[pl.BlockSpec(memory_space=pltpu.VMEM),
                  pl.BlockSpec(memory_space=pltpu.VMEM)],
        out_specs=pl.BlockSpec(memory_space=pltpu.SMEM),
    )(pos.reshape(128, 128), neg.reshape(B * K // 128, 128))


def kernel(center, context, negatives, center_weight, context_weight):
    center = center.astype(jnp.int32)
    context = context.astype(jnp.int32)
    negflat = negatives.astype(jnp.int32).reshape(B * K)
    pos, neg = _sc_dots(center, context, negflat, center_weight, context_weight)
    return _loss(pos, neg)[0, 0]


# speedup vs baseline: 1.0265x; 1.0265x over previous
import functools

import jax
import jax.numpy as jnp
from jax import lax
from jax.experimental import pallas as pl
from jax.experimental.pallas import tpu as pltpu
from jax.experimental.pallas import tpu_sc as plsc

B = 16384
K = 20
D = 16
NC = 2
NS = 16
NW = NC * NS
BPW = B // NW
C = 128
NCHUNK = BPW // C


def _sc_body(center_hbm, context_hbm, negflat_hbm, cw_hbm, ctw_hbm,
             pos_out, neg_out,
             cidx_v, pidx_v, nidx_v, vc_v, vp_v, vn_v, pos_v, neg_v,
             isem, csem, psem, nsem):
    wid = lax.axis_index("s") * NC + lax.axis_index("c")
    iota = lax.iota(jnp.int32, 16)

    for c in range(NCHUNK):
        base = wid * BPW + c * C

        pltpu.sync_copy(center_hbm.at[pl.ds(base, C)], cidx_v)
        pltpu.sync_copy(context_hbm.at[pl.ds(base, C)], pidx_v)
        idx_copies = [
            pltpu.async_copy(negflat_hbm.at[pl.ds(j * B + base, C)],
                             nidx_v.at[j], isem)
            for j in range(K)
        ]
        cc = pltpu.async_copy(cw_hbm.at[cidx_v], vc_v, csem)
        pc = pltpu.async_copy(ctw_hbm.at[pidx_v], vp_v, psem)
        for cp in idx_copies:
            cp.wait()
        ncopies = [
            pltpu.async_copy(ctw_hbm.at[nidx_v.at[j]],
                             vn_v.at[pl.ds(j * C, C)], nsem)
            for j in range(K)
        ]
        cc.wait()
        pc.wait()
        for cp in ncopies:
            cp.wait()

        def group(g, _):
            rows = iota + g * 16
            rowsk = rows * K
            pos_acc = jnp.zeros((16,), jnp.float32)
            neg_accs = [jnp.zeros((16,), jnp.float32) for _ in range(K)]
            for d in range(D):
                dsplat = jnp.full((16,), d, jnp.int32)
                vcg = plsc.load_gather(vc_v, [rows, dsplat])
                vpg = plsc.load_gather(vp_v, [rows, dsplat])
                pos_acc = pos_acc + vcg * vpg
                for k in range(K):
                    vng = plsc.load_gather(vn_v, [rows + k * C, dsplat])
                    neg_accs[k] = neg_accs[k] + vng * vcg
            pos_v[pl.ds(g * 16, 16)] = pos_acc
            for k in range(K):
                plsc.store_scatter(neg_v, [rowsk + k], neg_accs[k])
            return 0

        lax.fori_loop(0, C // 16, group, 0)

        pltpu.sync_copy(pos_v, pos_out.at[pl.ds(base, C)])
        pltpu.sync_copy(neg_v, neg_out.at[pl.ds(base * K, C * K)])


@jax.jit
def _sc_dots(center, context, negflat, cw, ctw):
    mesh = plsc.VectorSubcoreMesh(core_axis_name="c", subcore_axis_name="s")
    f = functools.partial(
        pl.kernel,
        out_type=(jax.ShapeDtypeStruct((B,), jnp.float32),
                  jax.ShapeDtypeStruct((B * K,), jnp.float32)),
        mesh=mesh,
        scratch_types=[
            pltpu.VMEM((C,), jnp.int32),
            pltpu.VMEM((C,), jnp.int32),
            pltpu.VMEM((K, C), jnp.int32),
            pltpu.VMEM((C, D), jnp.float32),
            pltpu.VMEM((C, D), jnp.float32),
            pltpu.VMEM((C * K, D), jnp.float32),
            pltpu.VMEM((C,), jnp.float32),
            pltpu.VMEM((C * K,), jnp.float32),
            pltpu.SemaphoreType.DMA,
            pltpu.SemaphoreType.DMA,
            pltpu.SemaphoreType.DMA,
            pltpu.SemaphoreType.DMA,
        ],
        compiler_params=pltpu.CompilerParams(needs_layout_passes=False,
                                             use_tc_tiling_on_sc=False),
    )(_sc_body)
    return f(center, context, negflat, cw, ctw)


def _loss_body(pos_ref, neg_ref, out_ref):
    pos = pos_ref[...]
    neg = neg_ref[...]
    sp = 1.0 / (1.0 + jnp.exp(-pos))
    sn = 1.0 / (1.0 + jnp.exp(neg))
    t1 = -jnp.sum(jnp.log(sp + 1e-8))
    t2 = -jnp.sum(jnp.log(sn + 1e-8))
    out_ref[0, 0] = (t1 + t2) / B


@jax.jit
def _loss(pos, neg):
    return pl.pallas_call(
        _loss_body,
        out_shape=jax.ShapeDtypeStruct((1, 1), jnp.float32),
        in_specs=[pl.BlockSpec(memory_space=pltpu.VMEM),
                  pl.BlockSpec(memory_space=pltpu.VMEM)],
        out_specs=pl.BlockSpec(memory_space=pltpu.SMEM),
    )(pos.reshape(128, 128), neg.reshape(B * K // 128, 128))


def kernel(center, context, negatives, center_weight, context_weight):
    center = center.astype(jnp.int32)
    context = context.astype(jnp.int32)
    negflat = negatives.astype(jnp.int32).T.reshape(K * B)
    pos, neg = _sc_dots(center, context, negflat, center_weight, context_weight)
    return _loss(pos, neg)[0, 0]
